# Initial kernel scaffold; baseline (speedup 1.0000x reference)
import jax
import jax.numpy as jnp
from jax.experimental import pallas as pl


def kernel(xyz, feats, W, bias):
    raise NotImplementedError("write your pallas kernel here")


# speedup vs baseline: 10.8392x; 10.8392x over previous
import functools

import jax
import jax.numpy as jnp
import numpy as np
from jax.experimental import pallas as pl
from jax.experimental.pallas import tpu as pltpu

BS = 8; NPTS = 2048; NA = 12
DIM_IN = 32; DIM_OUT = 32; KS = 3
APERTURE = 1.6; SIGMA = 0.2; ANN = 3

ROWS_IN = DIM_IN * NA
ROWS_OUT = DIM_OUT * NA
PT = 2048


def _anchor_mix_matrix():
    phi = (1.0 + np.sqrt(5.0)) / 2.0
    verts = []
    for s1 in (-1.0, 1.0):
        for s2 in (-1.0, 1.0):
            verts.append([0.0, s1, s2 * phi])
            verts.append([s1, s2 * phi, 0.0])
            verts.append([s2 * phi, 0.0, s1])
    v = np.asarray(verts, dtype=np.float32)
    v = v / np.linalg.norm(v, axis=1, keepdims=True)
    anchors = jnp.asarray(v[:NA])
    kernels = jnp.linspace(0.0, APERTURE, KS)
    dots = jnp.clip(anchors @ anchors.T, -1.0, 1.0)
    dists = jnp.arccos(dots)
    diff = dists[:, None, :] - kernels[None, :, None]
    w = jnp.exp(-(diff ** 2) / (2.0 * SIGMA))
    w = jnp.where(dists[:, None, :] <= APERTURE + 1e-6, w, 0.0)
    topw, idx = jax.lax.top_k(w, ANN)
    topw = topw / (jnp.sum(topw, axis=-1, keepdims=True) + 1e-9)
    onehot = jax.nn.one_hot(idx, NA, dtype=jnp.float32)
    return jnp.einsum('okaj,oka->koj', onehot, topw)


def _zpconv_kernel(wm_ref, bias_ref, x_ref, o_ref):
    x = x_ref[0]
    acc = jax.lax.dot_general(
        wm_ref[...], x, (((1,), (0,)), ((), ())),
        preferred_element_type=jnp.float32)
    o_ref[0] = acc + bias_ref[...]


@jax.jit
def kernel(xyz, feats, W, bias):
    del xyz
    M = _anchor_mix_matrix()
    Wr = W.reshape(DIM_OUT, DIM_IN, KS)
    WM = jnp.einsum('uck,koj->uocj', Wr, M).reshape(ROWS_OUT, ROWS_IN)
    bias_col = jnp.repeat(bias[0, :, 0], NA)[:, None]

    ft = feats.transpose(0, 1, 3, 2).reshape(BS, ROWS_IN, NPTS)

    n_pt = NPTS // PT
    out = pl.pallas_call(
        _zpconv_kernel,
        grid=(BS, n_pt),
        in_specs=[
            pl.BlockSpec((ROWS_OUT, ROWS_IN), lambda b, p: (0, 0)),
            pl.BlockSpec((ROWS_OUT, 1), lambda b, p: (0, 0)),
            pl.BlockSpec((1, ROWS_IN, PT), lambda b, p: (b, 0, p)),
        ],
        out_specs=pl.BlockSpec((1, ROWS_OUT, PT), lambda b, p: (b, 0, p)),
        out_shape=jax.ShapeDtypeStruct((BS, ROWS_OUT, NPTS), jnp.float32),
        compiler_params=pltpu.CompilerParams(
            dimension_semantics=("parallel", "parallel")),
    )(WM, bias_col, ft)

    return out.reshape(BS, DIM_OUT, NA, NPTS).transpose(0, 1, 3, 2)


# speedup vs baseline: 50.5758x; 4.6660x over previous
import functools

import jax
import jax.numpy as jnp
import numpy as np
from jax.experimental import pallas as pl
from jax.experimental.pallas import tpu as pltpu

BS = 8; NPTS = 2048; NA = 12
DIM_IN = 32; DIM_OUT = 32; KS = 3
APERTURE = 1.6; SIGMA = 0.2; ANN = 3

ROWS_IN = DIM_IN * NA
ROWS_OUT = DIM_OUT * NA
PT = 2048


def _anchor_mix_matrix():
    phi = (1.0 + np.sqrt(5.0)) / 2.0
    verts = []
    for s1 in (-1.0, 1.0):
        for s2 in (-1.0, 1.0):
            verts.append([0.0, s1, s2 * phi])
            verts.append([s1, s2 * phi, 0.0])
            verts.append([s2 * phi, 0.0, s1])
    v = np.asarray(verts, dtype=np.float32)
    v = v / np.linalg.norm(v, axis=1, keepdims=True)
    anchors = jnp.asarray(v[:NA])
    kernels = jnp.linspace(0.0, APERTURE, KS)
    dots = jnp.clip(anchors @ anchors.T, -1.0, 1.0)
    dists = jnp.arccos(dots)
    diff = dists[:, None, :] - kernels[None, :, None]
    w = jnp.exp(-(diff ** 2) / (2.0 * SIGMA))
    w = jnp.where(dists[:, None, :] <= APERTURE + 1e-6, w, 0.0)
    topw, idx = jax.lax.top_k(w, ANN)
    topw = topw / (jnp.sum(topw, axis=-1, keepdims=True) + 1e-9)
    onehot = jax.nn.one_hot(idx, NA, dtype=jnp.float32)
    return jnp.einsum('okaj,oka->koj', onehot, topw)


def _zpconv_kernel(wm_ref, bias_ref, x_ref, o_ref):
    x = x_ref[0]
    acc = jax.lax.dot_general(
        wm_ref[...], x, (((1,), (0,)), ((), ())),
        preferred_element_type=jnp.float32)
    o_ref[0] = acc + bias_ref[...]


@jax.jit
def kernel(xyz, feats, W, bias):
    del xyz
    M = _anchor_mix_matrix()
    Wr = W.reshape(DIM_OUT, DIM_IN, KS)
    WM = jnp.einsum('uck,koj->oujc', Wr, M).reshape(ROWS_OUT, ROWS_IN)
    bias_col = jnp.tile(bias[0, :, 0], NA)[:, None]

    ft = feats.transpose(0, 3, 1, 2).reshape(BS, ROWS_IN, NPTS)

    n_pt = NPTS // PT
    out = pl.pallas_call(
        _zpconv_kernel,
        grid=(BS, n_pt),
        in_specs=[
            pl.BlockSpec((ROWS_OUT, ROWS_IN), lambda b, p: (0, 0)),
            pl.BlockSpec((ROWS_OUT, 1), lambda b, p: (0, 0)),
            pl.BlockSpec((1, ROWS_IN, PT), lambda b, p: (b, 0, p)),
        ],
        out_specs=pl.BlockSpec((1, ROWS_OUT, PT), lambda b, p: (b, 0, p)),
        out_shape=jax.ShapeDtypeStruct((BS, ROWS_OUT, NPTS), jnp.float32),
        compiler_params=pltpu.CompilerParams(
            dimension_semantics=("parallel", "parallel")),
    )(WM, bias_col, ft)

    return out.reshape(BS, NA, DIM_OUT, NPTS).transpose(0, 2, 3, 1)


# speedup vs baseline: 55.6595x; 1.1005x over previous
import functools

import jax
import jax.numpy as jnp
import numpy as np
from jax.experimental import pallas as pl
from jax.experimental.pallas import tpu as pltpu

BS = 8; NPTS = 2048; NA = 12
DIM_IN = 32; DIM_OUT = 32; KS = 3
APERTURE = 1.6; SIGMA = 0.2; ANN = 3

ROWS_IN = DIM_IN * NA
ROWS_OUT = DIM_OUT * NA
PT = 2048


def _anchor_mix_matrix():
    phi = (1.0 + np.sqrt(5.0)) / 2.0
    verts = []
    for s1 in (-1.0, 1.0):
        for s2 in (-1.0, 1.0):
            verts.append([0.0, s1, s2 * phi])
            verts.append([s1, s2 * phi, 0.0])
            verts.append([s2 * phi, 0.0, s1])
    v = np.asarray(verts, dtype=np.float32)
    anchors = (v / np.linalg.norm(v, axis=1, keepdims=True))[:NA]
    kernels = np.linspace(0.0, APERTURE, KS, dtype=np.float32)
    dots = np.clip(anchors @ anchors.T, -1.0, 1.0).astype(np.float32)
    dists = np.arccos(dots).astype(np.float32)
    diff = dists[:, None, :] - kernels[None, :, None]
    w = np.exp(-(diff.astype(np.float32) ** 2) / np.float32(2.0 * SIGMA))
    w = np.where(dists[:, None, :] <= APERTURE + 1e-6, w.astype(np.float32),
                 np.float32(0.0))
    idx = np.argsort(-w, axis=-1, kind='stable')[..., :ANN]
    topw = np.take_along_axis(w, idx, axis=-1).astype(np.float32)
    topw = (topw / (topw.sum(-1, keepdims=True) + np.float32(1e-9)))
    M = np.zeros((KS, NA, NA), np.float32)
    o_i, k_i, _ = np.meshgrid(np.arange(NA), np.arange(KS), np.arange(ANN),
                              indexing='ij')
    np.add.at(M, (k_i, o_i, idx), topw.astype(np.float32))
    return M


_M_CONST = _anchor_mix_matrix()


def _zpconv_kernel(wm_ref, bias_ref, x_ref, o_ref):
    x = x_ref[0]
    acc = jax.lax.dot_general(
        wm_ref[...], x, (((1,), (0,)), ((), ())),
        preferred_element_type=jnp.float32)
    o_ref[0] = acc + bias_ref[...]


@jax.jit
def kernel(xyz, feats, W, bias):
    del xyz
    M = jnp.asarray(_M_CONST)
    Wr = W.reshape(DIM_OUT, DIM_IN, KS)
    WM = jnp.einsum('uck,koj->oujc', Wr, M).reshape(ROWS_OUT, ROWS_IN)
    bias_col = jnp.tile(bias[0, :, 0], NA)[:, None]

    ft = feats.transpose(0, 3, 1, 2).reshape(BS, ROWS_IN, NPTS)

    n_pt = NPTS // PT
    out = pl.pallas_call(
        _zpconv_kernel,
        grid=(BS, n_pt),
        in_specs=[
            pl.BlockSpec((ROWS_OUT, ROWS_IN), lambda b, p: (0, 0)),
            pl.BlockSpec((ROWS_OUT, 1), lambda b, p: (0, 0)),
            pl.BlockSpec((1, ROWS_IN, PT), lambda b, p: (b, 0, p)),
        ],
        out_specs=pl.BlockSpec((1, ROWS_OUT, PT), lambda b, p: (b, 0, p)),
        out_shape=jax.ShapeDtypeStruct((BS, ROWS_OUT, NPTS), jnp.float32),
        compiler_params=pltpu.CompilerParams(
            dimension_semantics=("parallel", "parallel")),
    )(WM, bias_col, ft)

    return out.reshape(BS, NA, DIM_OUT, NPTS).transpose(0, 2, 3, 1)
